# Initial kernel scaffold; baseline (speedup 1.0000x reference)
import jax
import jax.numpy as jnp
from jax.experimental import pallas as pl


def kernel(atom_num, table):
    raise NotImplementedError("write your pallas kernel here")


# speedup vs baseline: 1.3078x; 1.3078x over previous
import functools

import jax
import jax.numpy as jnp
from jax import lax
from jax.experimental import pallas as pl
from jax.experimental.pallas import tpu as pltpu
from jax.experimental.pallas import tpu_sc as plsc

HIDDEN_DIM = 128
N = 100000

_NC = 2
_NS = 16
_NW = _NC * _NS

_N_PAD = 100352
_PER_W = _N_PAD // _NW
_CHUNK = 392
_NCHUNK = _PER_W // _CHUNK


@functools.partial(
    pl.kernel,
    mesh=plsc.VectorSubcoreMesh(core_axis_name="c", subcore_axis_name="s"),
    out_type=jax.ShapeDtypeStruct((_N_PAD, HIDDEN_DIM), jnp.float32),
    scratch_types=[
        pltpu.VMEM((_CHUNK,), jnp.int32),
        pltpu.VMEM((_CHUNK, HIDDEN_DIM), jnp.float32),
        pltpu.SemaphoreType.DMA,
    ],
)
def _embedding_gather(table_hbm, idx_hbm, out_hbm, idx_v, rows_v, sem):
    wid = lax.axis_index("s") * _NC + lax.axis_index("c")
    base = wid * _PER_W

    def body(c, carry):
        off = base + c * _CHUNK
        pltpu.sync_copy(idx_hbm.at[pl.ds(off, _CHUNK)], idx_v)
        pltpu.async_copy(table_hbm.at[idx_v], rows_v, sem).wait()
        pltpu.sync_copy(rows_v, out_hbm.at[pl.ds(off, _CHUNK)])
        return carry

    lax.fori_loop(0, _NCHUNK, body, 0)


def kernel(atom_num, table):
    idx = atom_num.astype(jnp.int32)
    idx = jnp.concatenate([idx, jnp.zeros((_N_PAD - N,), jnp.int32)])
    out = _embedding_gather(table, idx)
    return out[:N]


# speedup vs baseline: 1.6678x; 1.2753x over previous
import functools

import jax
import jax.numpy as jnp
from jax import lax
from jax.experimental import pallas as pl
from jax.experimental.pallas import tpu as pltpu
from jax.experimental.pallas import tpu_sc as plsc

HIDDEN_DIM = 128
N = 100000

_NC = 2
_NS = 16
_NW = _NC * _NS

_PER_W = 3136
_CHUNK = 392
_NCHUNK = _PER_W // _CHUNK
_TAIL = N - (_NW - 1) * _PER_W - (_NCHUNK - 1) * _CHUNK


@functools.partial(
    pl.kernel,
    mesh=plsc.VectorSubcoreMesh(core_axis_name="c", subcore_axis_name="s"),
    out_type=jax.ShapeDtypeStruct((N, HIDDEN_DIM), jnp.float32),
    scratch_types=[
        pltpu.VMEM((_CHUNK,), jnp.int32),
        pltpu.VMEM((_CHUNK, HIDDEN_DIM), jnp.float32),
        pltpu.SemaphoreType.DMA,
    ],
)
def _embedding_gather(table_hbm, idx_hbm, out_hbm, idx_v, rows_v, sem):
    wid = lax.axis_index("s") * _NC + lax.axis_index("c")
    base = wid * _PER_W

    def full_chunk(off):
        pltpu.sync_copy(idx_hbm.at[pl.ds(off, _CHUNK)], idx_v)
        pltpu.async_copy(table_hbm.at[idx_v], rows_v, sem).wait()
        pltpu.sync_copy(rows_v, out_hbm.at[pl.ds(off, _CHUNK)])

    def body(c, carry):
        full_chunk(base + c * _CHUNK)
        return carry

    lax.fori_loop(0, _NCHUNK - 1, body, 0)

    last_off = base + (_NCHUNK - 1) * _CHUNK

    @pl.when(wid < _NW - 1)
    def _():
        full_chunk(last_off)

    @pl.when(wid == _NW - 1)
    def _():
        pltpu.sync_copy(idx_hbm.at[pl.ds(last_off, _TAIL)],
                        idx_v.at[pl.ds(0, _TAIL)])
        pltpu.async_copy(table_hbm.at[idx_v.at[pl.ds(0, _TAIL)]],
                         rows_v.at[pl.ds(0, _TAIL)], sem).wait()
        pltpu.sync_copy(rows_v.at[pl.ds(0, _TAIL)],
                        out_hbm.at[pl.ds(last_off, _TAIL)])


def kernel(atom_num, table):
    idx = atom_num.astype(jnp.int32)
    return _embedding_gather(table, idx)


# speedup vs baseline: 1.6713x; 1.0021x over previous
import functools

import jax
import jax.numpy as jnp
from jax import lax
from jax.experimental import pallas as pl
from jax.experimental.pallas import tpu as pltpu
from jax.experimental.pallas import tpu_sc as plsc

HIDDEN_DIM = 128
N = 100000

_NC = 2
_NS = 16
_NW = _NC * _NS

_PER_W = 3136
_CHUNK = 392
_NCHUNK = _PER_W // _CHUNK
_TAIL = N - (_NW - 1) * _PER_W - (_NCHUNK - 1) * _CHUNK


@functools.partial(
    pl.kernel,
    mesh=plsc.VectorSubcoreMesh(core_axis_name="c", subcore_axis_name="s"),
    out_type=jax.ShapeDtypeStruct((N, HIDDEN_DIM), jnp.float32),
    scratch_types=[
        pltpu.VMEM((_CHUNK,), jnp.int32),
        pltpu.VMEM((_CHUNK,), jnp.int32),
        pltpu.VMEM((_CHUNK, HIDDEN_DIM), jnp.float32),
        pltpu.VMEM((_CHUNK, HIDDEN_DIM), jnp.float32),
        pltpu.SemaphoreType.DMA,
        pltpu.SemaphoreType.DMA,
        pltpu.SemaphoreType.DMA,
        pltpu.SemaphoreType.DMA,
    ],
)
def _embedding_gather(table_hbm, idx_hbm, out_hbm, idx0, idx1, rows0, rows1,
                      gsem0, gsem1, osem0, osem1):
    wid = lax.axis_index("s") * _NC + lax.axis_index("c")
    base = wid * _PER_W
    idx_v = (idx0, idx1)
    rows = (rows0, rows1)
    gsem = (gsem0, gsem1)
    osem = (osem0, osem1)

    def start_gather(k):
        b = k & 1
        pltpu.sync_copy(idx_hbm.at[pl.ds(base + k * _CHUNK, _CHUNK)],
                        idx_v[b])
        return pltpu.async_copy(table_hbm.at[idx_v[b]], rows[b], gsem[b])

    def start_store(k):
        b = k & 1
        return pltpu.async_copy(rows[b],
                                out_hbm.at[pl.ds(base + k * _CHUNK, _CHUNK)],
                                osem[b])

    g0 = start_gather(0)
    g1 = start_gather(1)
    gathers = [g0, g1]
    stores = [None, None]

    for k in range(_NCHUNK - 2):
        b = k & 1
        gathers[b].wait()
        stores[b] = start_store(k)
        if k + 2 < _NCHUNK - 1:
            stores[b].wait()
            gathers[b] = start_gather(k + 2)

    gathers[0].wait()
    stores[0] = start_store(_NCHUNK - 2)

    last_off = base + (_NCHUNK - 1) * _CHUNK
    stores[1].wait()

    @pl.when(wid < _NW - 1)
    def _():
        pltpu.sync_copy(idx_hbm.at[pl.ds(last_off, _CHUNK)], idx1)
        pltpu.async_copy(table_hbm.at[idx1], rows1, gsem1).wait()
        pltpu.async_copy(rows1, out_hbm.at[pl.ds(last_off, _CHUNK)],
                         osem1).wait()

    @pl.when(wid == _NW - 1)
    def _():
        pltpu.sync_copy(idx_hbm.at[pl.ds(last_off, _TAIL)],
                        idx1.at[pl.ds(0, _TAIL)])
        pltpu.async_copy(table_hbm.at[idx1.at[pl.ds(0, _TAIL)]],
                         rows1.at[pl.ds(0, _TAIL)], gsem1).wait()
        pltpu.async_copy(rows1.at[pl.ds(0, _TAIL)],
                         out_hbm.at[pl.ds(last_off, _TAIL)], osem1).wait()

    stores[0].wait()


def kernel(atom_num, table):
    idx = atom_num.astype(jnp.int32)
    return _embedding_gather(table, idx)


# speedup vs baseline: 5.2391x; 3.1348x over previous
import functools

import jax
import jax.numpy as jnp
from jax import lax
from jax.experimental import pallas as pl
from jax.experimental.pallas import tpu as pltpu
from jax.experimental.pallas import tpu_sc as plsc

HIDDEN_DIM = 128
VOCAB_ROWS = 118
N = 100000

_NC = 2
_NS = 16
_NW = _NC * _NS

_PER_W = 3136
_CHUNK = 392
_NCHUNK = _PER_W // _CHUNK
_TAIL = N - (_NW - 1) * _PER_W - (_NCHUNK - 1) * _CHUNK


@functools.partial(
    pl.kernel,
    mesh=plsc.VectorSubcoreMesh(core_axis_name="c", subcore_axis_name="s"),
    out_type=jax.ShapeDtypeStruct((N, HIDDEN_DIM), jnp.float32),
    scratch_types=[
        pltpu.VMEM((_CHUNK,), jnp.int32),
        pltpu.VMEM((_CHUNK,), jnp.int32),
        pltpu.VMEM((_CHUNK, HIDDEN_DIM), jnp.float32),
        pltpu.VMEM((_CHUNK, HIDDEN_DIM), jnp.float32),
        pltpu.VMEM_SHARED((VOCAB_ROWS, HIDDEN_DIM), jnp.float32),
        pltpu.SemaphoreType.DMA,
        pltpu.SemaphoreType.DMA,
        pltpu.SemaphoreType.DMA,
        pltpu.SemaphoreType.DMA,
    ],
)
def _embedding_gather(table_hbm, idx_hbm, out_hbm, idx0, idx1, rows0, rows1,
                      table_sh, gsem0, gsem1, osem0, osem1):
    wid = lax.axis_index("s") * _NC + lax.axis_index("c")
    base = wid * _PER_W
    idx_v = (idx0, idx1)
    rows = (rows0, rows1)
    gsem = (gsem0, gsem1)
    osem = (osem0, osem1)

    @pl.when(lax.axis_index("s") == 0)
    def _():
        pltpu.sync_copy(table_hbm, table_sh)

    plsc.subcore_barrier()

    def start_gather(k):
        b = k & 1
        pltpu.sync_copy(idx_hbm.at[pl.ds(base + k * _CHUNK, _CHUNK)],
                        idx_v[b])
        return pltpu.async_copy(table_sh.at[idx_v[b]], rows[b], gsem[b])

    def start_store(k):
        b = k & 1
        return pltpu.async_copy(rows[b],
                                out_hbm.at[pl.ds(base + k * _CHUNK, _CHUNK)],
                                osem[b])

    g0 = start_gather(0)
    g1 = start_gather(1)
    gathers = [g0, g1]
    stores = [None, None]

    for k in range(_NCHUNK - 2):
        b = k & 1
        gathers[b].wait()
        stores[b] = start_store(k)
        if k + 2 < _NCHUNK - 1:
            stores[b].wait()
            gathers[b] = start_gather(k + 2)

    gathers[0].wait()
    stores[0] = start_store(_NCHUNK - 2)

    last_off = base + (_NCHUNK - 1) * _CHUNK
    stores[1].wait()

    @pl.when(wid < _NW - 1)
    def _():
        pltpu.sync_copy(idx_hbm.at[pl.ds(last_off, _CHUNK)], idx1)
        pltpu.async_copy(table_sh.at[idx1], rows1, gsem1).wait()
        pltpu.async_copy(rows1, out_hbm.at[pl.ds(last_off, _CHUNK)],
                         osem1).wait()

    @pl.when(wid == _NW - 1)
    def _():
        pltpu.sync_copy(idx_hbm.at[pl.ds(last_off, _TAIL)],
                        idx1.at[pl.ds(0, _TAIL)])
        pltpu.async_copy(table_sh.at[idx1.at[pl.ds(0, _TAIL)]],
                         rows1.at[pl.ds(0, _TAIL)], gsem1).wait()
        pltpu.async_copy(rows1.at[pl.ds(0, _TAIL)],
                         out_hbm.at[pl.ds(last_off, _TAIL)], osem1).wait()

    stores[0].wait()


def kernel(atom_num, table):
    idx = atom_num.astype(jnp.int32)
    return _embedding_gather(table, idx)


# speedup vs baseline: 5.5594x; 1.0611x over previous
import functools

import jax
import jax.numpy as jnp
from jax import lax
from jax.experimental import pallas as pl
from jax.experimental.pallas import tpu as pltpu
from jax.experimental.pallas import tpu_sc as plsc

HIDDEN_DIM = 128
VOCAB_ROWS = 118
N = 100000

_NC = 2
_NS = 16
_NW = _NC * _NS

_PER_W = 3136
_CHUNK = 392
_NCHUNK = _PER_W // _CHUNK
_PER_W_LAST = N - (_NW - 1) * _PER_W
_TAIL = _PER_W_LAST - (_NCHUNK - 1) * _CHUNK


@functools.partial(
    pl.kernel,
    mesh=plsc.VectorSubcoreMesh(core_axis_name="c", subcore_axis_name="s"),
    out_type=jax.ShapeDtypeStruct((N, HIDDEN_DIM), jnp.float32),
    scratch_types=[
        pltpu.VMEM((_PER_W,), jnp.int32),
        pltpu.VMEM((_CHUNK, HIDDEN_DIM), jnp.float32),
        pltpu.VMEM((_CHUNK, HIDDEN_DIM), jnp.float32),
        pltpu.VMEM_SHARED((VOCAB_ROWS, HIDDEN_DIM), jnp.float32),
        pltpu.SemaphoreType.DMA,
        pltpu.SemaphoreType.DMA,
        pltpu.SemaphoreType.DMA,
        pltpu.SemaphoreType.DMA,
    ],
)
def _embedding_gather(table_hbm, idx_hbm, out_hbm, idx_all, rows0, rows1,
                      table_sh, gsem0, gsem1, osem0, osem1):
    wid = lax.axis_index("s") * _NC + lax.axis_index("c")
    base = wid * _PER_W
    rows = (rows0, rows1)
    gsem = (gsem0, gsem1)
    osem = (osem0, osem1)

    @pl.when(lax.axis_index("s") == 0)
    def _():
        pltpu.sync_copy(table_hbm, table_sh)

    @pl.when(wid < _NW - 1)
    def _():
        pltpu.sync_copy(idx_hbm.at[pl.ds(base, _PER_W)], idx_all)

    @pl.when(wid == _NW - 1)
    def _():
        pltpu.sync_copy(idx_hbm.at[pl.ds(base, _PER_W_LAST)],
                        idx_all.at[pl.ds(0, _PER_W_LAST)])

    plsc.subcore_barrier()

    def chunk(k, nrows, b):
        pltpu.async_copy(
            table_sh.at[idx_all.at[pl.ds(k * _CHUNK, nrows)]],
            rows[b].at[pl.ds(0, nrows)], gsem[b]).wait()
        return pltpu.async_copy(
            rows[b].at[pl.ds(0, nrows)],
            out_hbm.at[pl.ds(base + k * _CHUNK, nrows)], osem[b])

    stores = [None, None]
    for k in range(_NCHUNK - 1):
        b = k & 1
        if stores[b] is not None:
            stores[b].wait()
        stores[b] = chunk(k, _CHUNK, b)

    stores[1].wait()

    @pl.when(wid < _NW - 1)
    def _():
        chunk(_NCHUNK - 1, _CHUNK, 1).wait()

    @pl.when(wid == _NW - 1)
    def _():
        chunk(_NCHUNK - 1, _TAIL, 1).wait()

    stores[0].wait()


def kernel(atom_num, table):
    idx = atom_num.astype(jnp.int32)
    return _embedding_gather(table, idx)
